# baseline (speedup 1.0000x reference)
import jax
import jax.numpy as jnp
from jax.experimental import pallas as pl
from jax.experimental.pallas import tpu as pltpu

DIM_K = 2048
NG = 16
GS = 4096
CAP = 320
TB1 = 512
NB1 = GS // TB1
TB = 128
NB = GS // TB
EPS_ = 1e-9
NEG_BIG = -3.4e38


def _body(x_ref, w_ref, disp_ref, comb_ref, loss_ref, c1_ref, c2_ref,
          sm_s, acc_ref, acc2_ref):
    i = pl.program_id(0)

    @pl.when(i == 0)
    def _init():
        acc_ref[...] = jnp.zeros_like(acc_ref)
        acc2_ref[...] = jnp.zeros_like(acc2_ref)

    @pl.when(i < NB1)
    def _phase1():
        raw = jnp.dot(x_ref[...], w_ref[...],
                      preferred_element_type=jnp.float32)
        m = jnp.max(raw, axis=1, keepdims=True)
        e = jnp.exp(raw - m)
        sm = e / jnp.sum(e, axis=1, keepdims=True)
        sm_s[pl.ds(i * TB1, TB1), :] = sm
        g1 = jnp.max(sm, axis=1, keepdims=True)
        iota = jax.lax.broadcasted_iota(jnp.int32, (TB1, NG), 1)
        i1 = jnp.min(jnp.where(sm == g1, iota, NG), axis=1, keepdims=True)
        mask1 = (iota == i1).astype(jnp.float32)
        acc2_ref[0:1, :] += jnp.sum(mask1, axis=0, keepdims=True)
        acc2_ref[1:2, :] += jnp.sum(sm, axis=0, keepdims=True)

    @pl.when(i >= NB1)
    def _phase2():
        j = i - NB1
        cnt = acc2_ref[0:1, :]
        smt = jnp.transpose(sm_s[pl.ds(j * TB, TB), :])
        eiota = jax.lax.broadcasted_iota(jnp.int32, (NG, TB), 0)
        g1 = jnp.max(smt, axis=0, keepdims=True)
        i1 = jnp.min(jnp.where(smt == g1, eiota, NG), axis=0, keepdims=True)
        masked = jnp.where(eiota == i1, NEG_BIG, smt)
        g2 = jnp.max(masked, axis=0, keepdims=True)
        i2 = jnp.min(jnp.where(masked == g2, eiota, NG), axis=0,
                     keepdims=True)
        mask1 = (eiota == i1).astype(jnp.float32)
        mask2 = (eiota == i2).astype(jnp.float32)

        r = jax.lax.broadcasted_iota(jnp.int32, (TB, TB), 0)
        c = jax.lax.broadcasted_iota(jnp.int32, (TB, TB), 1)
        triu = (r < c).astype(jnp.float32)
        prev1 = jnp.dot(mask1, triu, preferred_element_type=jnp.float32)
        prev2 = jnp.dot(mask2, triu, preferred_element_type=jnp.float32)

        c1_run = acc_ref[:, 0:1]
        c2_run = acc_ref[:, 1:2]
        m1cnt = jnp.minimum(jnp.transpose(cnt), float(CAP))

        pos1 = jnp.sum((c1_run + prev1) * mask1, axis=0, keepdims=True)
        keep1 = (pos1 < float(CAP)).astype(jnp.float32)
        pos2 = jnp.sum((c2_run + prev2 + m1cnt) * mask2, axis=0,
                       keepdims=True)
        keep2 = (pos2 < float(CAP)).astype(jnp.float32)

        acc_ref[:, 0:1] += jnp.sum(mask1, axis=1, keepdims=True)
        acc_ref[:, 1:2] += jnp.sum(mask2, axis=1, keepdims=True)
        acc_ref[:, 2:3] += jnp.sum(mask2 * keep2, axis=1, keepdims=True)

        denom = g1 + g2 + EPS_
        g1k = (g1 / denom) * keep1
        g2k = (g2 / denom) * keep2
        d1 = (g1k != 0.0).astype(jnp.float32)
        d2 = (g2k != 0.0).astype(jnp.float32)

        i1f = i1.astype(jnp.float32)
        i2f = i2.astype(jnp.float32)
        piota = jax.lax.broadcasted_iota(jnp.int32, (CAP, TB), 0).astype(
            jnp.float32)
        for e in range(NG):
            ef = float(e)
            is1 = i1f == ef
            is2 = i2f == ef
            pos_e = jnp.where(is1, pos1, jnp.where(is2, pos2, -1.0))
            val_e = jnp.where(is1, g1k, jnp.where(is2, g2k, 0.0))
            dva_e = jnp.where(is1, d1, jnp.where(is2, d2, 0.0))
            b = piota == pos_e
            comb_ref[e] = jnp.where(b, val_e, 0.0)
            disp_ref[e] = jnp.where(b, dva_e, 0.0)

        @pl.when(j == NB - 1)
        def _fin():
            c1_ref[...] = jnp.minimum(cnt, float(CAP))
            c2_ref[...] = jnp.transpose(acc_ref[:, 2:3])
            loss_ref[...] = jnp.sum(cnt * acc2_ref[1:2, :], axis=1,
                                    keepdims=True) * (
                                        float(NG) / (float(GS) * float(GS)))


def kernel(x, w_gating):
    x2 = x.reshape(GS, DIM_K)
    disp_t, comb_t, loss, c1, c2 = pl.pallas_call(
        _body,
        grid=(NB1 + NB,),
        in_specs=[
            pl.BlockSpec((TB1, DIM_K),
                         lambda i: (jnp.minimum(i, NB1 - 1), 0)),
            pl.BlockSpec((DIM_K, NG), lambda i: (0, 0)),
        ],
        out_specs=[
            pl.BlockSpec((NG, CAP, TB),
                         lambda i: (0, 0, jnp.maximum(i - NB1, 0))),
            pl.BlockSpec((NG, CAP, TB),
                         lambda i: (0, 0, jnp.maximum(i - NB1, 0))),
            pl.BlockSpec((1, 1), lambda i: (0, 0)),
            pl.BlockSpec((1, NG), lambda i: (0, 0)),
            pl.BlockSpec((1, NG), lambda i: (0, 0)),
        ],
        out_shape=[
            jax.ShapeDtypeStruct((NG, CAP, GS), jnp.float32),
            jax.ShapeDtypeStruct((NG, CAP, GS), jnp.float32),
            jax.ShapeDtypeStruct((1, 1), jnp.float32),
            jax.ShapeDtypeStruct((1, NG), jnp.float32),
            jax.ShapeDtypeStruct((1, NG), jnp.float32),
        ],
        scratch_shapes=[
            pltpu.VMEM((GS, NG), jnp.float32),
            pltpu.VMEM((NG, 8), jnp.float32),
            pltpu.VMEM((2, NG), jnp.float32),
        ],
        compiler_params=pltpu.CompilerParams(
            dimension_semantics=("arbitrary",)),
    )(x2, w_gating)

    disp = jnp.transpose(disp_t, (2, 0, 1))[None]
    comb = jnp.transpose(comb_t, (2, 0, 1))[None]
    return (disp, comb, loss[0, 0], c1, c2)


# speedup vs baseline: 1.0371x; 1.0371x over previous
import jax
import jax.numpy as jnp
from jax.experimental import pallas as pl
from jax.experimental.pallas import tpu as pltpu

DIM_K = 2048
NG = 16
GS = 4096
CAP = 320
TB1 = 1024
NB1 = GS // TB1
TB = 256
NB = GS // TB
EPS_ = 1e-9
NEG_BIG = -3.4e38


def _body(x_ref, w_ref, disp_ref, comb_ref, loss_ref, c1_ref, c2_ref,
          sm_s, acc_ref, acc2_ref):
    i = pl.program_id(0)

    @pl.when(i == 0)
    def _init():
        acc_ref[...] = jnp.zeros_like(acc_ref)
        acc2_ref[...] = jnp.zeros_like(acc2_ref)

    @pl.when(i < NB1)
    def _phase1():
        raw = jnp.dot(x_ref[...], w_ref[...],
                      preferred_element_type=jnp.float32)
        m = jnp.max(raw, axis=1, keepdims=True)
        e = jnp.exp(raw - m)
        sm = e / jnp.sum(e, axis=1, keepdims=True)
        sm_s[pl.ds(i * TB1, TB1), :] = sm
        g1 = jnp.max(sm, axis=1, keepdims=True)
        iota = jax.lax.broadcasted_iota(jnp.int32, (TB1, NG), 1)
        i1 = jnp.min(jnp.where(sm == g1, iota, NG), axis=1, keepdims=True)
        mask1 = (iota == i1).astype(jnp.float32)
        acc2_ref[0:1, :] += jnp.sum(mask1, axis=0, keepdims=True)
        acc2_ref[1:2, :] += jnp.sum(sm, axis=0, keepdims=True)

    @pl.when(i >= NB1)
    def _phase2():
        j = i - NB1
        cnt = acc2_ref[0:1, :]
        smt = jnp.transpose(sm_s[pl.ds(j * TB, TB), :])
        eiota = jax.lax.broadcasted_iota(jnp.int32, (NG, TB), 0)
        g1 = jnp.max(smt, axis=0, keepdims=True)
        i1 = jnp.min(jnp.where(smt == g1, eiota, NG), axis=0, keepdims=True)
        masked = jnp.where(eiota == i1, NEG_BIG, smt)
        g2 = jnp.max(masked, axis=0, keepdims=True)
        i2 = jnp.min(jnp.where(masked == g2, eiota, NG), axis=0,
                     keepdims=True)
        mask1 = (eiota == i1).astype(jnp.float32)
        mask2 = (eiota == i2).astype(jnp.float32)

        r = jax.lax.broadcasted_iota(jnp.int32, (TB, TB), 0)
        c = jax.lax.broadcasted_iota(jnp.int32, (TB, TB), 1)
        triu = (r < c).astype(jnp.float32)
        prev1 = jnp.dot(mask1, triu, preferred_element_type=jnp.float32)
        prev2 = jnp.dot(mask2, triu, preferred_element_type=jnp.float32)

        c1_run = acc_ref[:, 0:1]
        c2_run = acc_ref[:, 1:2]
        m1cnt = jnp.minimum(jnp.transpose(cnt), float(CAP))

        pos1 = jnp.sum((c1_run + prev1) * mask1, axis=0, keepdims=True)
        keep1 = (pos1 < float(CAP)).astype(jnp.float32)
        pos2 = jnp.sum((c2_run + prev2 + m1cnt) * mask2, axis=0,
                       keepdims=True)
        keep2 = (pos2 < float(CAP)).astype(jnp.float32)

        acc_ref[:, 0:1] += jnp.sum(mask1, axis=1, keepdims=True)
        acc_ref[:, 1:2] += jnp.sum(mask2, axis=1, keepdims=True)
        acc_ref[:, 2:3] += jnp.sum(mask2 * keep2, axis=1, keepdims=True)

        denom = g1 + g2 + EPS_
        g1k = (g1 / denom) * keep1
        g2k = (g2 / denom) * keep2
        d1 = (g1k != 0.0).astype(jnp.float32)
        d2 = (g2k != 0.0).astype(jnp.float32)

        i1f = i1.astype(jnp.float32)
        i2f = i2.astype(jnp.float32)
        piota = jax.lax.broadcasted_iota(jnp.int32, (CAP, TB), 0).astype(
            jnp.float32)
        for e in range(NG):
            ef = float(e)
            is1 = i1f == ef
            is2 = i2f == ef
            pos_e = jnp.where(is1, pos1, jnp.where(is2, pos2, -1.0))
            val_e = jnp.where(is1, g1k, jnp.where(is2, g2k, 0.0))
            dva_e = jnp.where(is1, d1, jnp.where(is2, d2, 0.0))
            b = piota == pos_e
            comb_ref[e] = jnp.where(b, val_e, 0.0)
            disp_ref[e] = jnp.where(b, dva_e, 0.0)

        @pl.when(j == NB - 1)
        def _fin():
            c1_ref[...] = jnp.minimum(cnt, float(CAP))
            c2_ref[...] = jnp.transpose(acc_ref[:, 2:3])
            loss_ref[...] = jnp.sum(cnt * acc2_ref[1:2, :], axis=1,
                                    keepdims=True) * (
                                        float(NG) / (float(GS) * float(GS)))


def kernel(x, w_gating):
    x2 = x.reshape(GS, DIM_K)
    disp_t, comb_t, loss, c1, c2 = pl.pallas_call(
        _body,
        grid=(NB1 + NB,),
        in_specs=[
            pl.BlockSpec((TB1, DIM_K),
                         lambda i: (jnp.minimum(i, NB1 - 1), 0)),
            pl.BlockSpec((DIM_K, NG), lambda i: (0, 0)),
        ],
        out_specs=[
            pl.BlockSpec((NG, CAP, TB),
                         lambda i: (0, 0, jnp.maximum(i - NB1, 0))),
            pl.BlockSpec((NG, CAP, TB),
                         lambda i: (0, 0, jnp.maximum(i - NB1, 0))),
            pl.BlockSpec((1, 1), lambda i: (0, 0)),
            pl.BlockSpec((1, NG), lambda i: (0, 0)),
            pl.BlockSpec((1, NG), lambda i: (0, 0)),
        ],
        out_shape=[
            jax.ShapeDtypeStruct((NG, CAP, GS), jnp.float32),
            jax.ShapeDtypeStruct((NG, CAP, GS), jnp.float32),
            jax.ShapeDtypeStruct((1, 1), jnp.float32),
            jax.ShapeDtypeStruct((1, NG), jnp.float32),
            jax.ShapeDtypeStruct((1, NG), jnp.float32),
        ],
        scratch_shapes=[
            pltpu.VMEM((GS, NG), jnp.float32),
            pltpu.VMEM((NG, 8), jnp.float32),
            pltpu.VMEM((2, NG), jnp.float32),
        ],
        compiler_params=pltpu.CompilerParams(
            dimension_semantics=("arbitrary",)),
    )(x2, w_gating)

    disp = jnp.transpose(disp_t, (2, 0, 1))[None]
    comb = jnp.transpose(comb_t, (2, 0, 1))[None]
    return (disp, comb, loss[0, 0], c1, c2)
